# Initial kernel scaffold; baseline (speedup 1.0000x reference)
import jax
import jax.numpy as jnp
from jax.experimental import pallas as pl


def kernel(full_features, sp_coords, full_super_indices_10, full_super_indices_21, W1, b1, W2, b2, W3, b3, W4, b4, Wp1, bp1, Wp2, bp2):
    raise NotImplementedError("write your pallas kernel here")


# speedup vs baseline: 2.0512x; 2.0512x over previous
import functools

import jax
import jax.numpy as jnp
from jax.experimental import pallas as pl
from jax.experimental.pallas import tpu as pltpu

N = 16384
S1 = 512
S2 = 16
PAD = 64
EMB = 384

TILE = 1024
NT = N // TILE

_NINF = float("-inf")


def _seg_scan_fwd(m, seg, cs, cv):
    r = m.shape[0]
    d = 1
    while d < r:
        m_sh = jnp.concatenate(
            [jnp.full((d, m.shape[1]), _NINF, m.dtype), m[: r - d]], axis=0)
        s_sh = jnp.concatenate(
            [jnp.full((d, 1), -1, seg.dtype), seg[: r - d]], axis=0)
        m = jnp.maximum(m, jnp.where(s_sh == seg, m_sh, _NINF))
        d *= 2
    m = jnp.maximum(m, jnp.where(seg == cs, cv, _NINF))
    return m


def _seg_scan_bwd(m, seg, cs, cv):
    r = m.shape[0]
    d = 1
    while d < r:
        m_sh = jnp.concatenate(
            [m[d:], jnp.full((d, m.shape[1]), _NINF, m.dtype)], axis=0)
        s_sh = jnp.concatenate(
            [seg[d:], jnp.full((d, 1), -1, seg.dtype)], axis=0)
        m = jnp.maximum(m, jnp.where(s_sh == seg, m_sh, _NINF))
        d *= 2
    m = jnp.maximum(m, jnp.where(seg == cs, cv, _NINF))
    return m


def _k1_body(x_ref, seg_ref, w1_ref, b1_ref, w2_ref, b2_ref,
             h_ref, fwd_ref, cs_ref, cv_ref):
    t = pl.program_id(0)

    @pl.when(t == 0)
    def _():
        cs_ref[...] = jnp.full((1, 1), -1, jnp.int32)
        cv_ref[...] = jnp.full(cv_ref.shape, _NINF, jnp.float32)

    h = jnp.dot(x_ref[...], w1_ref[...], preferred_element_type=jnp.float32)
    h = h + b1_ref[...]
    h = jnp.dot(h, w2_ref[...], preferred_element_type=jnp.float32)
    h = h + b2_ref[...]
    h_ref[...] = h
    seg = seg_ref[...]
    m = _seg_scan_fwd(h, seg, cs_ref[...], cv_ref[...])
    fwd_ref[...] = m
    cs_ref[...] = seg[TILE - 1:TILE, :]
    cv_ref[...] = m[TILE - 1:TILE, :]


def _k2_body(h_ref, fwd_ref, seg_ref, gb_ref, cs_ref, cv_ref):
    t = pl.program_id(0)

    @pl.when(t == 0)
    def _():
        cs_ref[...] = jnp.full((1, 1), -1, jnp.int32)
        cv_ref[...] = jnp.full(cv_ref.shape, _NINF, jnp.float32)

    seg = seg_ref[...]
    m = _seg_scan_bwd(h_ref[...], seg, cs_ref[...], cv_ref[...])
    gb_ref[...] = jnp.maximum(fwd_ref[...], m)
    cs_ref[...] = seg[0:1, :]
    cv_ref[...] = m[0:1, :]


def _k3_body(gb_ref, h_ref, seg_ref, w3a_ref, w3b_ref, b3_ref, w4_ref, b4_ref,
             pm_ref, cs_ref, cv_ref):
    t = pl.program_id(0)

    @pl.when(t == 0)
    def _():
        cs_ref[...] = jnp.full((1, 1), -1, jnp.int32)
        cv_ref[...] = jnp.full(cv_ref.shape, _NINF, jnp.float32)

    z = jnp.dot(gb_ref[...], w3a_ref[...], preferred_element_type=jnp.float32)
    z = z + jnp.dot(h_ref[...], w3b_ref[...], preferred_element_type=jnp.float32)
    z = jnp.maximum(z + b3_ref[...], 0.0)
    f2 = jnp.dot(z, w4_ref[...], preferred_element_type=jnp.float32)
    f2 = f2 + b4_ref[...]
    seg = seg_ref[...]
    m = _seg_scan_fwd(f2, seg, cs_ref[...], cv_ref[...])
    pm_ref[...] = m
    cs_ref[...] = seg[TILE - 1:TILE, :]
    cv_ref[...] = m[TILE - 1:TILE, :]


def _k4_body(pm_ref, sp_ref, wp1_ref, bp1_ref, wp2_ref, bp2_ref,
             ends_ref, emp_ref, dd_ref, vv_ref,
             tok_ref, pos_ref, pos_s):
    p = jnp.dot(sp_ref[...], wp1_ref[...], preferred_element_type=jnp.float32)
    p = p + bp1_ref[...]
    p = 0.5 * p * (1.0 + jax.lax.erf(p * 0.7071067811865476))
    p = jnp.dot(p, wp2_ref[...], preferred_element_type=jnp.float32)
    p = p + bp2_ref[...]
    pos_s[...] = p
    tok_ref[...] = jnp.zeros(tok_ref.shape, jnp.float32)
    pos_ref[...] = jnp.zeros(pos_ref.shape, jnp.float32)

    def it(s, carry):
        e = ends_ref[s]
        row = pm_ref[pl.ds(e, 1), :]
        row = jnp.where(emp_ref[s] == 1, _NINF, row)

        @pl.when(vv_ref[s] == 1)
        def _():
            d = dd_ref[s]
            tok_ref[pl.ds(d, 1), :] = row
            pos_ref[pl.ds(d, 1), :] = pos_s[pl.ds(s, 1), :]

        return carry

    jax.lax.fori_loop(0, S1, it, 0)


def kernel(full_features, sp_coords, full_super_indices_10,
           full_super_indices_21, W1, b1, W2, b2, W3, b3, W4, b4,
           Wp1, bp1, Wp2, bp2):
    f32 = jnp.float32
    i32 = jnp.int32
    seg10 = full_super_indices_10.astype(i32)
    seg21 = full_super_indices_21.astype(i32)
    seg_col = seg10.reshape(N, 1)

    counts10 = jnp.bincount(seg10, length=S1)
    ends10 = jnp.maximum(jnp.cumsum(counts10) - 1, 0).astype(i32)
    emp10 = (counts10 == 0).astype(i32)
    counts21 = jnp.bincount(seg21, length=S2)
    starts21 = jnp.concatenate(
        [jnp.zeros((1,), counts21.dtype), jnp.cumsum(counts21)[:-1]])
    rank = jnp.arange(S1, dtype=i32) - starts21[seg21].astype(i32)
    valid = rank < PAD
    dd = (seg21 * PAD + jnp.clip(rank, 0, PAD - 1)).astype(i32)
    vv = valid.astype(i32)

    w1t = W1.T
    w2t = W2.T
    w3t = W3.T
    w3a = w3t[:256]
    w3b = w3t[256:]
    w4t = W4.T
    wp1t = Wp1.T
    wp2t = Wp2.T
    b1r = b1.reshape(1, -1)
    b2r = b2.reshape(1, -1)
    b3r = b3.reshape(1, -1)
    b4r = b4.reshape(1, -1)
    bp1r = bp1.reshape(1, -1)
    bp2r = bp2.reshape(1, -1)

    full = lambda shape: pl.BlockSpec(shape, lambda t: (0,) * len(shape))
    row_blk = lambda c: pl.BlockSpec((TILE, c), lambda t: (t, 0))
    rev_blk = lambda c: pl.BlockSpec((TILE, c), lambda t: (NT - 1 - t, 0))

    h, fwd = pl.pallas_call(
        _k1_body,
        grid=(NT,),
        in_specs=[row_blk(6), row_blk(1), full((6, 128)), full((1, 128)),
                  full((128, 256)), full((1, 256))],
        out_specs=[row_blk(256), row_blk(256)],
        out_shape=[jax.ShapeDtypeStruct((N, 256), f32),
                   jax.ShapeDtypeStruct((N, 256), f32)],
        scratch_shapes=[pltpu.VMEM((1, 1), i32), pltpu.VMEM((1, 256), f32)],
    )(full_features, seg_col, w1t, b1r, w2t, b2r)

    gb = pl.pallas_call(
        _k2_body,
        grid=(NT,),
        in_specs=[rev_blk(256), rev_blk(256), rev_blk(1)],
        out_specs=rev_blk(256),
        out_shape=jax.ShapeDtypeStruct((N, 256), f32),
        scratch_shapes=[pltpu.VMEM((1, 1), i32), pltpu.VMEM((1, 256), f32)],
    )(h, fwd, seg_col)

    pm = pl.pallas_call(
        _k3_body,
        grid=(NT,),
        in_specs=[row_blk(256), row_blk(256), row_blk(1), full((256, 512)),
                  full((256, 512)), full((1, 512)), full((512, 384)),
                  full((1, 384))],
        out_specs=row_blk(384),
        out_shape=jax.ShapeDtypeStruct((N, 384), f32),
        scratch_shapes=[pltpu.VMEM((1, 1), i32), pltpu.VMEM((1, 384), f32)],
    )(gb, h, seg_col, w3a, w3b, b3r, w4t, b4r)

    smem = pl.BlockSpec(memory_space=pltpu.SMEM)
    tok_flat, pos_flat = pl.pallas_call(
        _k4_body,
        in_specs=[pl.BlockSpec(memory_space=pltpu.VMEM),
                  pl.BlockSpec(memory_space=pltpu.VMEM),
                  pl.BlockSpec(memory_space=pltpu.VMEM),
                  pl.BlockSpec(memory_space=pltpu.VMEM),
                  pl.BlockSpec(memory_space=pltpu.VMEM),
                  pl.BlockSpec(memory_space=pltpu.VMEM),
                  smem, smem, smem, smem],
        out_specs=[pl.BlockSpec(memory_space=pltpu.VMEM),
                   pl.BlockSpec(memory_space=pltpu.VMEM)],
        out_shape=[jax.ShapeDtypeStruct((S2 * PAD, EMB), f32),
                   jax.ShapeDtypeStruct((S2 * PAD, EMB), f32)],
        scratch_shapes=[pltpu.VMEM((S1, EMB), f32)],
    )(pm, sp_coords, wp1t, bp1r, wp2t, bp2r, ends10, emp10, dd, vv)

    tok_p = tok_flat.reshape(1, S2, PAD, EMB)
    pos_p = pos_flat.reshape(1, S2, PAD, EMB)
    return (tok_p, pos_p)
